# Initial kernel scaffold; baseline (speedup 1.0000x reference)
import jax
import jax.numpy as jnp
from jax.experimental import pallas as pl


def kernel(inputs_whole, width, height, derivative):
    raise NotImplementedError("write your pallas kernel here")


# speedup vs baseline: 7.5798x; 7.5798x over previous
import functools

import jax
import jax.numpy as jnp
from jax.experimental import pallas as pl
from jax.experimental.pallas import tpu as pltpu

N = 4194304
LEFT, RIGHT, BOTTOM, TOP = -0.5, 0.5, -0.5, 0.5
MIN_BIN_WIDTH = 1e-3
MIN_BIN_HEIGHT = 1e-3
MIN_DERIVATIVE = 1e-3

ROWS = 4096
COLS = 1024
BLOCK_ROWS = 256


def _spline_elementwise(x, wraw, hraw, draw):
    inside = jnp.logical_and(x > LEFT, x < RIGHT)
    xi = jnp.clip(x, LEFT + 1e-6, RIGHT - 1e-6)

    w = (1.0 / (1.0 + jnp.exp(-wraw))) * (1.0 - 2.0 * MIN_BIN_WIDTH) + MIN_BIN_WIDTH
    h = (1.0 / (1.0 + jnp.exp(-hraw))) * (1.0 - 2.0 * MIN_BIN_HEIGHT) + MIN_BIN_HEIGHT
    d = jnp.exp(draw) * (1.0 - MIN_DERIVATIVE) + MIN_DERIVATIVE

    in1 = xi >= (w - 0.5)
    icw = jnp.where(in1, w - 0.5, LEFT)
    ibw = jnp.where(in1, 1.0 - w, w)
    ich = jnp.where(in1, h - 0.5, BOTTOM)
    ih = jnp.where(in1, 1.0 - h, h)
    idelta = ih / ibw
    id0 = jnp.where(in1, d, 1.0)
    id1 = jnp.where(in1, 1.0, d)

    theta = (xi - icw) / ibw
    tt = theta * (1.0 - theta)
    num = ih * (idelta * theta * theta + id0 * tt)
    den = idelta + (id0 + id1 - 2.0 * idelta) * tt
    out = ich + num / den
    dnum = (idelta * idelta) * (
        id1 * theta * theta + 2.0 * idelta * tt + id0 * (1.0 - theta) * (1.0 - theta)
    )
    lad = jnp.log(dnum) - 2.0 * jnp.log(den)
    out = jnp.clip(out, BOTTOM, TOP)

    return jnp.where(inside, out, x), jnp.where(inside, lad, 0.0)


def _tc_body(x_ref, w_ref, h_ref, d_ref, out_ref, lad_ref):
    o, l = _spline_elementwise(x_ref[...], w_ref[...], h_ref[...], d_ref[...])
    out_ref[...] = o
    lad_ref[...] = l


@jax.jit
def kernel(inputs_whole, width, height, derivative):
    x = inputs_whole.reshape(ROWS, COLS)
    w = width.reshape(ROWS, COLS)
    h = height.reshape(ROWS, COLS)
    d = derivative.reshape(ROWS, COLS)

    bs = pl.BlockSpec((BLOCK_ROWS, COLS), lambda i: (i, 0))
    out, lad = pl.pallas_call(
        _tc_body,
        grid=(ROWS // BLOCK_ROWS,),
        in_specs=[bs, bs, bs, bs],
        out_specs=[bs, bs],
        out_shape=[
            jax.ShapeDtypeStruct((ROWS, COLS), jnp.float32),
            jax.ShapeDtypeStruct((ROWS, COLS), jnp.float32),
        ],
    )(x, w, h, d)
    return out.reshape(N), lad.reshape(N)


# speedup vs baseline: 92.8966x; 12.2558x over previous
import functools

import jax
import jax.numpy as jnp
from jax.experimental import pallas as pl
from jax.experimental.pallas import tpu as pltpu

N = 4194304
LEFT, RIGHT, BOTTOM, TOP = -0.5, 0.5, -0.5, 0.5
MIN_BIN_WIDTH = 1e-3
MIN_BIN_HEIGHT = 1e-3
MIN_DERIVATIVE = 1e-3

ROWS = 32768
COLS = 128
BLOCK_ROWS = 2048


def _spline_elementwise(x, wraw, hraw, draw):
    inside = jnp.logical_and(x > LEFT, x < RIGHT)
    xi = jnp.clip(x, LEFT + 1e-6, RIGHT - 1e-6)

    w = (1.0 / (1.0 + jnp.exp(-wraw))) * (1.0 - 2.0 * MIN_BIN_WIDTH) + MIN_BIN_WIDTH
    h = (1.0 / (1.0 + jnp.exp(-hraw))) * (1.0 - 2.0 * MIN_BIN_HEIGHT) + MIN_BIN_HEIGHT
    d = jnp.exp(draw) * (1.0 - MIN_DERIVATIVE) + MIN_DERIVATIVE

    in1 = xi >= (w - 0.5)
    icw = jnp.where(in1, w - 0.5, LEFT)
    ibw = jnp.where(in1, 1.0 - w, w)
    ich = jnp.where(in1, h - 0.5, BOTTOM)
    ih = jnp.where(in1, 1.0 - h, h)
    rib = 1.0 / ibw
    idelta = ih * rib
    id0 = jnp.where(in1, d, 1.0)
    id1 = jnp.where(in1, 1.0, d)

    theta = (xi - icw) * rib
    omt = 1.0 - theta
    tt = theta * omt
    num = ih * (idelta * theta * theta + id0 * tt)
    den = idelta + (id0 + id1 - 2.0 * idelta) * tt
    rden = 1.0 / den
    out = ich + num * rden
    dnum = (idelta * idelta) * (
        id1 * theta * theta + 2.0 * idelta * tt + id0 * omt * omt
    )
    lad = jnp.log(dnum * rden * rden)
    out = jnp.clip(out, BOTTOM, TOP)

    return jnp.where(inside, out, x), jnp.where(inside, lad, 0.0)


def _tc_body(x_ref, w_ref, h_ref, d_ref, out_ref, lad_ref):
    o, l = _spline_elementwise(x_ref[...], w_ref[...], h_ref[...], d_ref[...])
    out_ref[...] = o
    lad_ref[...] = l


@jax.jit
def kernel(inputs_whole, width, height, derivative):
    x = inputs_whole.reshape(ROWS, COLS)
    w = width.reshape(ROWS, COLS)
    h = height.reshape(ROWS, COLS)
    d = derivative.reshape(ROWS, COLS)

    bs = pl.BlockSpec((BLOCK_ROWS, COLS), lambda i: (i, 0))
    out, lad = pl.pallas_call(
        _tc_body,
        grid=(ROWS // BLOCK_ROWS,),
        in_specs=[bs, bs, bs, bs],
        out_specs=[bs, bs],
        out_shape=[
            jax.ShapeDtypeStruct((ROWS, COLS), jnp.float32),
            jax.ShapeDtypeStruct((ROWS, COLS), jnp.float32),
        ],
    )(x, w, h, d)
    return out.reshape(N), lad.reshape(N)
